# Initial kernel scaffold; baseline (speedup 1.0000x reference)
import jax
import jax.numpy as jnp
from jax.experimental import pallas as pl


def kernel(input_tensor):
    raise NotImplementedError("write your pallas kernel here")


# speedup vs baseline: 12.7511x; 12.7511x over previous
import functools

import jax
import jax.numpy as jnp
from jax.experimental import pallas as pl

_ROWS_PER_BLOCK = 8
_NEWTON_ITERS = 24


def _sparsemax_block(x_ref, o_ref):
    x = x_ref[...]
    m = jnp.max(x, axis=1, keepdims=True)
    t0 = m - 1.0

    def body(_, t):
        active = x > t
        s = jnp.sum(jnp.where(active, x, 0.0), axis=1, keepdims=True)
        k = jnp.sum(jnp.where(active, 1.0, 0.0), axis=1, keepdims=True)
        return (s - 1.0) / k

    tau = jax.lax.fori_loop(0, _NEWTON_ITERS, body, t0)
    o_ref[...] = jnp.maximum(x - tau, 0.0)


@jax.jit
def kernel(input_tensor):
    rows, n = input_tensor.shape
    grid = (rows // _ROWS_PER_BLOCK,)
    return pl.pallas_call(
        _sparsemax_block,
        grid=grid,
        in_specs=[pl.BlockSpec((_ROWS_PER_BLOCK, n), lambda i: (i, 0))],
        out_specs=pl.BlockSpec((_ROWS_PER_BLOCK, n), lambda i: (i, 0)),
        out_shape=jax.ShapeDtypeStruct((rows, n), input_tensor.dtype),
    )(input_tensor)


# speedup vs baseline: 24.2236x; 1.8997x over previous
import functools

import jax
import jax.numpy as jnp
from jax.experimental import pallas as pl

_ROWS_PER_BLOCK = 8
_NEWTON_ITERS = 12


def _sparsemax_block(x_ref, o_ref):
    x = x_ref[...]
    m = jnp.max(x, axis=1, keepdims=True)
    t0 = m - 1.0

    def body(_, t):
        active = x > t
        s = jnp.sum(jnp.where(active, x, 0.0), axis=1, keepdims=True)
        k = jnp.sum(jnp.where(active, 1.0, 0.0), axis=1, keepdims=True)
        return (s - 1.0) / k

    tau = jax.lax.fori_loop(0, _NEWTON_ITERS, body, t0)
    o_ref[...] = jnp.maximum(x - tau, 0.0)


@jax.jit
def kernel(input_tensor):
    rows, n = input_tensor.shape
    grid = (rows // _ROWS_PER_BLOCK,)
    return pl.pallas_call(
        _sparsemax_block,
        grid=grid,
        in_specs=[pl.BlockSpec((_ROWS_PER_BLOCK, n), lambda i: (i, 0))],
        out_specs=pl.BlockSpec((_ROWS_PER_BLOCK, n), lambda i: (i, 0)),
        out_shape=jax.ShapeDtypeStruct((rows, n), input_tensor.dtype),
    )(input_tensor)
